# baseline (speedup 1.0000x reference)
import functools
import itertools

import jax
import jax.numpy as jnp
from jax import lax
from jax.experimental import pallas as pl
from jax.experimental.pallas import tpu as pltpu

K_GEN = 4
LOSS_SUBSET = 0.25
W_LABELS = 1.0
W_TIME = 1.0

_LB = 128


def _loss_kernel(rw_ref, time_ref, lab_ref, len_ref, pt_ref, pl_ref,
                 out_ref, scr_ref, acc_ref, cnt_ref,
                 *, L, B, K, C, n_idx, nb):
    i = pl.program_id(0)
    s = i * _LB

    lens = len_ref[0:1, :]

    @pl.when(i == 0)
    def _init():
        acc_ref[0] = jnp.float32(0.0)
        cnt_ref[0] = jnp.int32(0)
        rw = rw_ref[...]
        pos = lax.broadcasted_iota(jnp.int32, (L, B), 0)
        validpos = (pos + K_GEN) < lens
        bits = lax.bitcast_convert_type(rw, jnp.int32)
        bits = jnp.where(validpos, bits, 0)

        def body(j, prefix):
            cand = prefix | (jnp.int32(1) << (30 - j))
            cnt = jnp.sum((bits >= cand).astype(jnp.int32), axis=0,
                          keepdims=True)
            return jnp.where(cnt >= n_idx, cand, prefix)

        tau = lax.fori_loop(0, 31, body, jnp.zeros((1, B), jnp.int32))
        cnt_gt = jnp.sum((bits > tau).astype(jnp.int32), axis=0,
                         keepdims=True)
        m = n_idx - cnt_gt
        scr_ref[0:1, 0:B] = tau
        scr_ref[1:2, 0:B] = m
        scr_ref[2:3, 0:B] = jnp.zeros((1, B), jnp.int32)

    posb = lax.broadcasted_iota(jnp.int32, (_LB, B), 0) + s
    validb = (posb + K_GEN) < lens
    bitsb = lax.bitcast_convert_type(rw_ref[pl.ds(s, _LB), :], jnp.int32)
    bitsb = jnp.where(validb, bitsb, 0)
    tau = scr_ref[0:1, 0:B]
    m = scr_ref[1:2, 0:B]
    eqc = scr_ref[2:3, 0:B]
    gt = bitsb > tau
    eq = bitsb == tau
    eqf = eq.astype(jnp.float32)
    rr = lax.broadcasted_iota(jnp.int32, (_LB, _LB), 0)
    cc = lax.broadcasted_iota(jnp.int32, (_LB, _LB), 1)
    tri = (cc < rr).astype(jnp.float32)
    pref = jax.lax.dot(tri, eqf,
                       preferred_element_type=jnp.float32)
    rank = pref.astype(jnp.int32) + eqc
    contrib = (gt | (eq & (rank < m))) & validb
    scr_ref[2:3, 0:B] = eqc + jnp.sum(eq.astype(jnp.int32), axis=0,
                                      keepdims=True)

    logits3 = pl_ref[...]
    xs, logzs = [], []
    for kk in range(K):
        xk = logits3[:, :, kk * C:(kk + 1) * C]
        mxk = jnp.max(xk, axis=2, keepdims=True)
        logzk = mxk[..., 0] + jnp.log(jnp.sum(jnp.exp(xk - mxk), axis=2))
        xs.append(xk)
        logzs.append(logzk)

    s2 = lax.rem(s + _LB, L)
    tA = time_ref[pl.ds(s, _LB), :]
    tB = time_ref[pl.ds(s2, 8), :]
    twin = jnp.concatenate([tA, tB], axis=0)
    lA = lab_ref[pl.ds(s, _LB), :]
    lB = lab_ref[pl.ds(s2, 8), :]
    lwin = jnp.concatenate([lA, lB], axis=0)
    t0 = twin[0:_LB]
    ptb = pt_ref[...]

    cidx = lax.broadcasted_iota(jnp.int32, (_LB, B, C), 2)
    c = [[None] * K_GEN for _ in range(K)]
    for t in range(K_GEN):
        labt = lwin[1 + t:1 + t + _LB]
        oht = cidx == labt[:, :, None]
        dl = twin[1 + t:1 + t + _LB] - t0
        for kk in range(K):
            picked = jnp.sum(jnp.where(oht, xs[kk], 0.0), axis=2)
            ce = logzs[kk] - picked
            l1 = jnp.abs(ptb[:, :, kk] - dl)
            c[kk][t] = W_LABELS * ce + W_TIME * l1
    mc = None
    for p in itertools.permutations(range(K)):
        ps = c[0][p[0]] + c[1][p[1]] + c[2][p[2]] + c[3][p[3]]
        mc = ps if mc is None else jnp.minimum(mc, ps)

    maskf = contrib.astype(jnp.float32)
    acc_ref[0] += jnp.sum(mc * maskf)
    cnt_ref[0] += jnp.sum(contrib.astype(jnp.int32))

    @pl.when(i == nb - 1)
    def _fin():
        V = jnp.maximum(cnt_ref[0], 1)
        loss = acc_ref[0] / (V * K).astype(jnp.float32)
        out_ref[...] = jnp.full((1, 1), loss, jnp.float32)


@jax.jit
def kernel(time, labels, lengths, pred_time, pred_logits, rand_weights):
    L, B = time.shape
    K = pred_time.shape[2]
    C = pred_logits.shape[3]
    n_idx = min(max(int(round(L * LOSS_SUBSET)), 1), L)
    nb = L // _LB

    rw_t = rand_weights.T.astype(jnp.float32)
    lab_i = labels.astype(jnp.int32)
    len_b = jnp.broadcast_to(lengths.astype(jnp.int32)[None, :], (8, B))

    out = pl.pallas_call(
        functools.partial(_loss_kernel, L=L, B=B, K=K, C=C,
                          n_idx=n_idx, nb=nb),
        grid=(nb,),
        in_specs=[
            pl.BlockSpec((L, B), lambda i: (0, 0)),
            pl.BlockSpec((L, B), lambda i: (0, 0)),
            pl.BlockSpec((L, B), lambda i: (0, 0)),
            pl.BlockSpec((8, B), lambda i: (0, 0)),
            pl.BlockSpec((_LB, B, K), lambda i: (i, 0, 0)),
            pl.BlockSpec((_LB, B, K * C), lambda i: (i, 0, 0)),
        ],
        out_specs=pl.BlockSpec((1, 1), lambda i: (0, 0)),
        out_shape=jax.ShapeDtypeStruct((1, 1), jnp.float32),
        scratch_shapes=[
            pltpu.VMEM((8, 128), jnp.int32),
            pltpu.SMEM((1,), jnp.float32),
            pltpu.SMEM((1,), jnp.int32),
        ],
    )(rw_t, time.astype(jnp.float32), lab_i, len_b,
      pred_time.astype(jnp.float32),
      pred_logits.astype(jnp.float32).reshape(L, B, K * C))
    return out[0, 0]
